# baseline (speedup 1.0000x reference)
import functools

import jax
import jax.numpy as jnp
from jax import lax
from jax.experimental import pallas as pl
from jax.experimental.pallas import tpu as pltpu
from jax.experimental.pallas import tpu_sc as plsc

_B = 128
_C = 384
_S = 196
_D1 = 1024
_THRES = 0.3
_EPS = 1e-6
_NCHUNK = 13


def _sc_body(f1_hbm, f2_hbm, out_hbm, xbuf, row, sem):
    del sem
    ncores = 2
    wid = lax.axis_index("s") * ncores + lax.axis_index("c")
    lanes = lax.iota(jnp.int32, 16)
    tail_mask = lanes >= 12

    def accumulate(hbm_ref, b):
        pltpu.sync_copy(hbm_ref.at[b], xbuf)

        def cbody(c, accs):
            new = []
            for g in range(12):
                v = xbuf[c, pl.ds(16 * g, 16)]
                new.append(accs[g] + v * v)
            v = xbuf[c, pl.ds(180, 16)]
            new.append(accs[12] + jnp.where(tail_mask, v * v, 0.0))
            return tuple(new)

        zero = jnp.zeros((16,), jnp.float32)
        return lax.fori_loop(0, _C, cbody, tuple(zero for _ in range(_NCHUNK)))

    pos_inf = jnp.float32(jnp.inf)
    neg_inf = jnp.float32(-jnp.inf)

    for kk in range(_B // 32):
        b = wid * (_B // 32) + kk
        a1 = accumulate(f1_hbm, b)
        a2 = accumulate(f2_hbm, b)

        dsq = jnp.zeros((16,), jnp.float32)
        for g in range(_NCHUNK):
            d = a1[g] - a2[g]
            dsq = dsq + d * d
        mse_raw = jnp.sum(dsq)

        mx1v = a1[0]
        mn1v = a1[0]
        mx2v = a2[0]
        mn2v = a2[0]
        for g in range(1, 12):
            mx1v = jnp.maximum(mx1v, a1[g])
            mn1v = jnp.minimum(mn1v, a1[g])
            mx2v = jnp.maximum(mx2v, a2[g])
            mn2v = jnp.minimum(mn2v, a2[g])
        mx1v = jnp.maximum(mx1v, jnp.where(tail_mask, a1[12], neg_inf))
        mn1v = jnp.minimum(mn1v, jnp.where(tail_mask, a1[12], pos_inf))
        mx2v = jnp.maximum(mx2v, jnp.where(tail_mask, a2[12], neg_inf))
        mn2v = jnp.minimum(mn2v, jnp.where(tail_mask, a2[12], pos_inf))

        inv_c = jnp.float32(1.0 / _C)
        mse_b = mse_raw * (inv_c * inv_c)
        mx1 = jnp.max(mx1v) * inv_c
        mn1 = jnp.min(mn1v) * inv_c
        mx2 = jnp.max(mx2v) * inv_c
        mn2 = jnp.min(mn2v) * inv_c

        r = jnp.where(lanes == 0, mse_b,
                      jnp.where(lanes == 1, mx1,
                                jnp.where(lanes == 2, mn1,
                                          jnp.where(lanes == 3, mx2,
                                                    jnp.where(lanes == 4, mn2,
                                                              0.0)))))
        row[pl.ds(0, 16)] = r
        if kk == 0:
            z = jnp.zeros((16,), jnp.float32)
            for t in range(1, 8):
                row[pl.ds(16 * t, 16)] = z
        pltpu.sync_copy(row, out_hbm.at[b])


def _sc_partials(f1, f2):
    mesh = plsc.VectorSubcoreMesh(core_axis_name="c", subcore_axis_name="s")
    return pl.kernel(
        _sc_body,
        out_type=jax.ShapeDtypeStruct((_B, 128), jnp.float32),
        mesh=mesh,
        scratch_types=[
            pltpu.VMEM((_C, _S), jnp.float32),
            pltpu.VMEM((128,), jnp.float32),
            pltpu.SemaphoreType.DMA,
        ],
        compiler_params=pltpu.CompilerParams(needs_layout_passes=False),
    )(f1, f2)


def _tc_body(part_ref, a_ref, b_ref, q1c_ref, q2c_ref, w_ref,
             loss_ref, wl_ref):
    q1c = q1c_ref[...]
    q2c = q2c_ref[...]
    qm = jnp.abs(q1c - q2c)
    mean_q = jnp.sum(qm) * (1.0 / _B)
    margin_upper = 100.0 - (100.0 - mean_q) * _THRES
    margin_lower = mean_q * _THRES
    maskq = (qm < margin_lower) | (qm > margin_upper)
    q1z = jnp.where(maskq, 0.0, q1c)
    q2z = jnp.where(maskq, 0.0, q2c)
    f1h = q1z > q2z
    act = q1z != q2z
    count = jnp.sum(act.astype(jnp.float32))
    sum_q1 = jnp.sum(q1z)

    part = part_ref[...]
    mse_b = part[:, 0:1]
    mx1 = part[:, 1:2]
    mn1 = part[:, 2:3]
    mx2 = part[:, 3:4]
    mn2 = part[:, 4:5]
    fs_mx = jnp.where(f1h, mx2, mx1)
    fs_mn = jnp.where(f1h, mn2, mn1)
    mse_sum = jnp.sum(jnp.where(act, mse_b, 0.0))
    fs_max = jnp.max(jnp.where(act, fs_mx, -jnp.inf))
    fs_min = jnp.min(jnp.where(act, fs_mn, jnp.inf))

    a = a_ref[...]
    b = b_ref[...]
    dot = jnp.sum(a * b, axis=1, keepdims=True)
    na = jnp.sqrt(jnp.sum(a * a, axis=1, keepdims=True))
    nb = jnp.sqrt(jnp.sum(b * b, axis=1, keepdims=True))
    denom = jnp.maximum(na, _EPS) * jnp.maximum(nb, _EPS)
    cd = 1.0 - dot / denom
    cos_sum = jnp.sum(jnp.where(act, cd, 0.0))

    mse_loss = mse_sum / (count * jnp.float32(_S))
    ampify = 2.0 / (fs_max - fs_min)
    loss0 = ampify * mse_loss
    loss1 = cos_sum / count

    w0 = w_ref[0]
    w1 = w_ref[1]
    wl0 = loss0 * w0
    wl1 = loss1 * w1
    loss_all = wl0 + wl1

    zero_case = sum_q1 == 0.0
    loss_all = jnp.where(zero_case, 0.0, loss_all)
    wl0 = jnp.where(zero_case, 0.0, wl0)
    wl1 = jnp.where(zero_case, 0.0, wl1)

    loss_ref[...] = jnp.full((1, 1), loss_all)
    wl_ref[...] = jnp.concatenate(
        [jnp.full((1, 1), wl0), jnp.full((1, 1), wl1)], axis=1)


def _tc_finalize(part, f1l1, f2l1, q1c, q2c, w):
    out = pl.pallas_call(
        _tc_body,
        in_specs=[
            pl.BlockSpec((_B, 128), lambda: (0, 0)),
            pl.BlockSpec((_B, _D1), lambda: (0, 0)),
            pl.BlockSpec((_B, _D1), lambda: (0, 0)),
            pl.BlockSpec((_B, 1), lambda: (0, 0)),
            pl.BlockSpec((_B, 1), lambda: (0, 0)),
            pl.BlockSpec(memory_space=pltpu.SMEM),
        ],
        out_specs=[
            pl.BlockSpec((1, 1), lambda: (0, 0)),
            pl.BlockSpec((1, 2), lambda: (0, 0)),
        ],
        out_shape=[
            jax.ShapeDtypeStruct((1, 1), jnp.float32),
            jax.ShapeDtypeStruct((1, 2), jnp.float32),
        ],
    )(part, f1l1, f2l1, q1c, q2c, w)
    return out[0].reshape(()), out[1].reshape(2)


@jax.jit
def _qsd_loss(f1l0, f1l1, f2l0, f2l1, q1, q2, w):
    f1 = f1l0.reshape(_B, _C, _S)
    f2 = f2l0.reshape(_B, _C, _S)
    part = _sc_partials(f1, f2)
    return _tc_finalize(part, f1l1, f2l1,
                        q1.reshape(_B, 1), q2.reshape(_B, 1), w)


def kernel(features_1_level0, features_1_level1, features_2_level0,
           features_2_level1, quality_1, quality_2, weights):
    return _qsd_loss(features_1_level0, features_1_level1,
                     features_2_level0, features_2_level1,
                     quality_1, quality_2, weights)


# speedup vs baseline: 1.0027x; 1.0027x over previous
import functools

import jax
import jax.numpy as jnp
from jax import lax
from jax.experimental import pallas as pl
from jax.experimental.pallas import tpu as pltpu
from jax.experimental.pallas import tpu_sc as plsc

_B = 128
_C = 384
_S = 196
_D1 = 1024
_THRES = 0.3
_EPS = 1e-6
_NCHUNK = 13


def _sc_body(f1_hbm, f2_hbm, out_hbm, xbuf, row, sem):
    del sem
    ncores = 2
    wid = lax.axis_index("s") * ncores + lax.axis_index("c")
    lanes = lax.iota(jnp.int32, 16)
    tail_mask = lanes >= 12

    def accumulate(hbm_ref, b):
        pltpu.sync_copy(hbm_ref.at[b], xbuf)

        def cbody(c, accs):
            new = []
            for g in range(12):
                v = xbuf[c, pl.ds(16 * g, 16)]
                new.append(accs[g] + v * v)
            v = xbuf[c, pl.ds(180, 16)]
            new.append(accs[12] + jnp.where(tail_mask, v * v, 0.0))
            return tuple(new)

        zero = jnp.zeros((16,), jnp.float32)
        return lax.fori_loop(0, _C, cbody, tuple(zero for _ in range(_NCHUNK)))

    pos_inf = jnp.float32(jnp.inf)
    neg_inf = jnp.float32(-jnp.inf)

    for kk in range(_B // 32):
        b = wid * (_B // 32) + kk
        a1 = accumulate(f1_hbm, b)
        a2 = accumulate(f2_hbm, b)

        dsq = jnp.zeros((16,), jnp.float32)
        for g in range(_NCHUNK):
            d = a1[g] - a2[g]
            dsq = dsq + d * d
        mse_raw = jnp.sum(dsq)

        mx1v = a1[0]
        mn1v = a1[0]
        mx2v = a2[0]
        mn2v = a2[0]
        for g in range(1, 12):
            mx1v = jnp.maximum(mx1v, a1[g])
            mn1v = jnp.minimum(mn1v, a1[g])
            mx2v = jnp.maximum(mx2v, a2[g])
            mn2v = jnp.minimum(mn2v, a2[g])
        mx1v = jnp.maximum(mx1v, jnp.where(tail_mask, a1[12], neg_inf))
        mn1v = jnp.minimum(mn1v, jnp.where(tail_mask, a1[12], pos_inf))
        mx2v = jnp.maximum(mx2v, jnp.where(tail_mask, a2[12], neg_inf))
        mn2v = jnp.minimum(mn2v, jnp.where(tail_mask, a2[12], pos_inf))

        inv_c = jnp.float32(1.0 / _C)
        mse_b = mse_raw * (inv_c * inv_c)
        mx1 = jnp.max(mx1v) * inv_c
        mn1 = jnp.min(mn1v) * inv_c
        mx2 = jnp.max(mx2v) * inv_c
        mn2 = jnp.min(mn2v) * inv_c

        r = jnp.where(lanes == 0, mse_b,
                      jnp.where(lanes == 1, mx1,
                                jnp.where(lanes == 2, mn1,
                                          jnp.where(lanes == 3, mx2,
                                                    jnp.where(lanes == 4, mn2,
                                                              0.0)))))
        row[pl.ds(0, 16)] = r
        if kk == 0:
            z = jnp.zeros((16,), jnp.float32)
            for t in range(1, 8):
                row[pl.ds(16 * t, 16)] = z
        pltpu.sync_copy(row, out_hbm.at[b])


def _sc_partials(f1, f2):
    mesh = plsc.VectorSubcoreMesh(core_axis_name="c", subcore_axis_name="s")
    return pl.kernel(
        _sc_body,
        out_type=jax.ShapeDtypeStruct((_B, 128), jnp.float32),
        mesh=mesh,
        scratch_types=[
            pltpu.VMEM((_C, _S), jnp.float32),
            pltpu.VMEM((128,), jnp.float32),
            pltpu.SemaphoreType.DMA,
        ],
        compiler_params=pltpu.CompilerParams(needs_layout_passes=False,
                                             use_tc_tiling_on_sc=True),
    )(f1, f2)


def _tc_body(part_ref, a_ref, b_ref, q1c_ref, q2c_ref, w_ref,
             loss_ref, wl_ref):
    q1c = q1c_ref[...]
    q2c = q2c_ref[...]
    qm = jnp.abs(q1c - q2c)
    mean_q = jnp.sum(qm) * (1.0 / _B)
    margin_upper = 100.0 - (100.0 - mean_q) * _THRES
    margin_lower = mean_q * _THRES
    maskq = (qm < margin_lower) | (qm > margin_upper)
    q1z = jnp.where(maskq, 0.0, q1c)
    q2z = jnp.where(maskq, 0.0, q2c)
    f1h = q1z > q2z
    act = q1z != q2z
    count = jnp.sum(act.astype(jnp.float32))
    sum_q1 = jnp.sum(q1z)

    part = part_ref[...]
    mse_b = part[:, 0:1]
    mx1 = part[:, 1:2]
    mn1 = part[:, 2:3]
    mx2 = part[:, 3:4]
    mn2 = part[:, 4:5]
    fs_mx = jnp.where(f1h, mx2, mx1)
    fs_mn = jnp.where(f1h, mn2, mn1)
    mse_sum = jnp.sum(jnp.where(act, mse_b, 0.0))
    fs_max = jnp.max(jnp.where(act, fs_mx, -jnp.inf))
    fs_min = jnp.min(jnp.where(act, fs_mn, jnp.inf))

    a = a_ref[...]
    b = b_ref[...]
    dot = jnp.sum(a * b, axis=1, keepdims=True)
    na = jnp.sqrt(jnp.sum(a * a, axis=1, keepdims=True))
    nb = jnp.sqrt(jnp.sum(b * b, axis=1, keepdims=True))
    denom = jnp.maximum(na, _EPS) * jnp.maximum(nb, _EPS)
    cd = 1.0 - dot / denom
    cos_sum = jnp.sum(jnp.where(act, cd, 0.0))

    mse_loss = mse_sum / (count * jnp.float32(_S))
    ampify = 2.0 / (fs_max - fs_min)
    loss0 = ampify * mse_loss
    loss1 = cos_sum / count

    w0 = w_ref[0]
    w1 = w_ref[1]
    wl0 = loss0 * w0
    wl1 = loss1 * w1
    loss_all = wl0 + wl1

    zero_case = sum_q1 == 0.0
    loss_all = jnp.where(zero_case, 0.0, loss_all)
    wl0 = jnp.where(zero_case, 0.0, wl0)
    wl1 = jnp.where(zero_case, 0.0, wl1)

    loss_ref[...] = jnp.full((1, 1), loss_all)
    wl_ref[...] = jnp.concatenate(
        [jnp.full((1, 1), wl0), jnp.full((1, 1), wl1)], axis=1)


def _tc_finalize(part, f1l1, f2l1, q1c, q2c, w):
    out = pl.pallas_call(
        _tc_body,
        in_specs=[
            pl.BlockSpec((_B, 128), lambda: (0, 0)),
            pl.BlockSpec((_B, _D1), lambda: (0, 0)),
            pl.BlockSpec((_B, _D1), lambda: (0, 0)),
            pl.BlockSpec((_B, 1), lambda: (0, 0)),
            pl.BlockSpec((_B, 1), lambda: (0, 0)),
            pl.BlockSpec(memory_space=pltpu.SMEM),
        ],
        out_specs=[
            pl.BlockSpec((1, 1), lambda: (0, 0)),
            pl.BlockSpec((1, 2), lambda: (0, 0)),
        ],
        out_shape=[
            jax.ShapeDtypeStruct((1, 1), jnp.float32),
            jax.ShapeDtypeStruct((1, 2), jnp.float32),
        ],
    )(part, f1l1, f2l1, q1c, q2c, w)
    return out[0].reshape(()), out[1].reshape(2)


@jax.jit
def _qsd_loss(f1l0, f1l1, f2l0, f2l1, q1, q2, w):
    f1 = f1l0.reshape(_B, _C, _S)
    f2 = f2l0.reshape(_B, _C, _S)
    part = _sc_partials(f1, f2)
    return _tc_finalize(part, f1l1, f2l1,
                        q1.reshape(_B, 1), q2.reshape(_B, 1), w)


def kernel(features_1_level0, features_1_level1, features_2_level0,
           features_2_level1, quality_1, quality_2, weights):
    return _qsd_loss(features_1_level0, features_1_level1,
                     features_2_level0, features_2_level1,
                     quality_1, quality_2, weights)
